# baseline (speedup 1.0000x reference)
import jax
import jax.numpy as jnp
from jax import lax
from jax.experimental import pallas as pl
from jax.experimental.pallas import tpu as pltpu
from jax.experimental.pallas import tpu_sc as plsc

N = 10000
NU = 4000
D = 128
DH2 = 64
E_UI = 320000
E_UU = 128000
NC = 2
NS = 16
NW = NC * NS
C = 80
UI_PW = E_UI // NW
UU_PW = E_UU // NW
SCN = 25
SCE = SCN * C
UI_SC = UI_PW // SCE
UU_SC = UU_PW // SCE
NP = 10240
NUP = 4096
ROWS_UI = NP // NS
ROWS_UU = NUP // NS


def _spmm_body(xw_ui, xw_uu, zeros_hbm,
               ui_cols3, ui_rows3, ui_vals,
               uu_cols3, uu_rows3, uu_vals,
               out_ui, out_uu,
               cols_v, rows_v, vals_v, gath_v, acc, sem, ssem):
  c = lax.axis_index("c")
  s = lax.axis_index("s")
  w = c * NS + s

  pltpu.sync_copy(zeros_hbm, acc.at[pl.ds(s * ROWS_UI, ROWS_UI)])
  plsc.subcore_barrier()

  def run(xw_hbm, cols_hbm, rows_hbm, vals_hbm, acc, pw, nsc):
    def super_chunk(o, carry):
      pltpu.sync_copy(cols_hbm.at[w].at[o], cols_v)
      pltpu.sync_copy(rows_hbm.at[w].at[o], rows_v)
      pltpu.sync_copy(vals_hbm.at[pl.ds(w * pw + o * SCE, SCE)], vals_v)

      pltpu.async_copy(xw_hbm.at[pl.ds(0, C)], gath_v.at[0], sem)

      def chunk(i, carry2):
        b = lax.rem(i, 2)
        pltpu.make_async_copy(xw_hbm.at[pl.ds(0, C)], gath_v.at[b], sem).wait()

        @pl.when(i + 1 < SCN)
        def _():
          pltpu.async_copy(xw_hbm.at[pl.ds(0, C)], gath_v.at[1 - b], sem)

        return carry2

      lax.fori_loop(0, SCN, chunk, 0)
      return carry

    lax.fori_loop(0, nsc, super_chunk, 0)

  run(xw_ui, ui_cols3, ui_rows3, ui_vals, acc, UI_PW, UI_SC)
  plsc.subcore_barrier()
  pltpu.sync_copy(acc.at[pl.ds(s * ROWS_UI, ROWS_UI)],
                  out_ui.at[c].at[pl.ds(s * ROWS_UI, ROWS_UI)])
  plsc.subcore_barrier()

  pltpu.sync_copy(zeros_hbm.at[pl.ds(0, ROWS_UU)],
                  acc.at[pl.ds(s * ROWS_UU, ROWS_UU)])
  plsc.subcore_barrier()
  run(xw_uu, uu_cols3, uu_rows3, uu_vals, acc, UU_PW, UU_SC)
  plsc.subcore_barrier()
  pltpu.sync_copy(acc.at[pl.ds(s * ROWS_UU, ROWS_UU)],
                  out_uu.at[c].at[pl.ds(s * ROWS_UU, ROWS_UU)])


_spmm = pl.kernel(
    _spmm_body,
    out_type=[jax.ShapeDtypeStruct((NC, NP, D), jnp.float32),
              jax.ShapeDtypeStruct((NC, NUP, D), jnp.float32)],
    mesh=plsc.VectorSubcoreMesh(core_axis_name="c", subcore_axis_name="s"),
    scratch_types=[
        pltpu.VMEM((SCN, C), jnp.int32),
        pltpu.VMEM((SCN, C), jnp.int32),
        pltpu.VMEM((SCE,), jnp.float32),
        pltpu.VMEM((2, C, D), jnp.float32),
        pltpu.VMEM_SHARED((NP, D), jnp.float32),
        pltpu.SemaphoreType.DMA,
        pltpu.SemaphoreType.DMA,
    ],
)


def _mm_body(a_ref, w_ref, o_ref):
  o_ref[...] = jnp.dot(a_ref[...], w_ref[...], preferred_element_type=jnp.float32)


def _mm(a, w, bm):
  m, k = a.shape
  n = w.shape[1]
  return pl.pallas_call(
      _mm_body,
      grid=(m // bm,),
      in_specs=[pl.BlockSpec((bm, k), lambda i: (i, 0)),
                pl.BlockSpec((k, n), lambda i: (0, 0))],
      out_specs=pl.BlockSpec((bm, n), lambda i: (i, 0)),
      out_shape=jax.ShapeDtypeStruct((m, n), jnp.float32),
  )(a, w)


def _relu_mm_body(p_ref, w_ref, o_ref):
  x = jnp.maximum(p_ref[0] + p_ref[1], 0.0)
  o_ref[...] = jnp.dot(x, w_ref[...], preferred_element_type=jnp.float32)


def _relu_mm(p, w, bm):
  _, m, k = p.shape
  n = w.shape[1]
  return pl.pallas_call(
      _relu_mm_body,
      grid=(m // bm,),
      in_specs=[pl.BlockSpec((NC, bm, k), lambda i: (0, i, 0)),
                pl.BlockSpec((k, n), lambda i: (0, 0))],
      out_specs=pl.BlockSpec((bm, n), lambda i: (i, 0)),
      out_shape=jax.ShapeDtypeStruct((m, n), jnp.float32),
  )(p, w)


def _combine_body(p_ref, o_ref):
  o_ref[...] = p_ref[0] + p_ref[1]


def _combine(p, bm):
  _, m, k = p.shape
  return pl.pallas_call(
      _combine_body,
      grid=(m // bm,),
      in_specs=[pl.BlockSpec((NC, bm, k), lambda i: (0, i, 0))],
      out_specs=pl.BlockSpec((bm, k), lambda i: (i, 0)),
      out_shape=jax.ShapeDtypeStruct((m, k), jnp.float32),
  )(p)


def _dec_body(a_ref, b_ref, o_ref):
  o_ref[...] = lax.dot_general(a_ref[...], b_ref[...],
                               (((1,), (1,)), ((), ())),
                               preferred_element_type=jnp.float32)


def _dec(a, b, bm):
  m = a.shape[0]
  n = b.shape[0]
  k = a.shape[1]
  return pl.pallas_call(
      _dec_body,
      grid=(m // bm,),
      in_specs=[pl.BlockSpec((bm, k), lambda i: (i, 0)),
                pl.BlockSpec((n, k), lambda i: (0, 0))],
      out_specs=pl.BlockSpec((bm, n), lambda i: (i, 0)),
      out_shape=jax.ShapeDtypeStruct((m, n), jnp.float32),
  )(a, b)


def kernel(adj_ui_index, adj_ui_values, adj_uu_index, adj_uu_values,
           emb, W1_ui, W2_ui, W1_uu, W2_uu):
  ui_rows3 = adj_ui_index[0].reshape(NW, UI_SC, SCN, C)
  ui_cols3 = adj_ui_index[1].reshape(NW, UI_SC, SCN, C)
  uu_rows3 = adj_uu_index[0].reshape(NW, UU_SC, SCN, C)
  uu_cols3 = adj_uu_index[1].reshape(NW, UU_SC, SCN, C)
  zeros = jnp.zeros((ROWS_UI, D), jnp.float32)

  xw_ui = _mm(emb, W1_ui, 1000)
  xw_uu = _mm(emb[:NUP], W1_uu, 1024)
  p_ui, p_uu = _spmm(xw_ui, xw_uu, zeros,
                     ui_cols3, ui_rows3, adj_ui_values,
                     uu_cols3, uu_rows3, adj_uu_values)

  y_ui = _relu_mm(p_ui, W2_ui, 1024)
  y_uu = _relu_mm(p_uu, W2_uu, 1024)
  q_ui, q_uu = _spmm(y_ui, y_uu, zeros,
                     ui_cols3, ui_rows3, adj_ui_values,
                     uu_cols3, uu_rows3, adj_uu_values)

  h2_ui = _combine(q_ui, 1024)[:N]
  h2_uu = _combine(q_uu, 1024)[:NU]

  mu_ui = h2_ui[:, :DH2]
  logvar_ui = h2_ui[:, DH2:]
  mu_uu = h2_uu[:, :DH2]
  logvar_uu = h2_uu[:, DH2:]

  dec_ui = _dec(mu_ui[:NU], mu_ui[NU:], 400)
  dec_uu = _dec(mu_uu, mu_uu, 400)
  return (dec_ui, mu_ui, logvar_ui, dec_uu, mu_uu, logvar_uu)


# speedup vs baseline: 1.9741x; 1.9741x over previous
import jax
import jax.numpy as jnp
from jax import lax
from jax.experimental import pallas as pl
from jax.experimental.pallas import tpu as pltpu
from jax.experimental.pallas import tpu_sc as plsc

N = 10000
NU = 4000
D = 128
DH2 = 64
E_UI = 320000
E_UU = 128000
NC = 2
NS = 16
NW = NC * NS
C = 80
UI_PW = E_UI // NW
UU_PW = E_UU // NW
SCN = 25
SCE = SCN * C
UI_SC = UI_PW // SCE
UU_SC = UU_PW // SCE
NP = 10112
NUP = 4096
ROWS_UI = NP // NS
ROWS_UU = NUP // NS


def _spmm_body(xw_ui, xw_uu, zeros_hbm,
               ui_cols3, ui_rows3, ui_vals,
               uu_cols3, uu_rows3, uu_vals,
               out_ui, out_uu,
               cols_v, rows_v, vals_v, gath_v, acc, sem, ssem):
  c = lax.axis_index("c")
  s = lax.axis_index("s")
  w = c * NS + s

  pltpu.sync_copy(zeros_hbm, acc.at[pl.ds(s * ROWS_UI, ROWS_UI)])
  plsc.subcore_barrier()

  def run(xw_hbm, cols_hbm, rows_hbm, vals_hbm, acc, pw, nsc):
    def super_chunk(o, carry):
      pltpu.sync_copy(cols_hbm.at[w].at[o], cols_v)
      pltpu.sync_copy(rows_hbm.at[w].at[o], rows_v)
      pltpu.sync_copy(vals_hbm.at[pl.ds(w * pw + o * SCE, SCE)], vals_v)

      pltpu.async_copy(xw_hbm.at[cols_v.at[0]], gath_v.at[0], sem)
      pltpu.async_copy(xw_hbm.at[cols_v.at[1]], gath_v.at[1], sem)

      def chunk(i, carry2):
        b = lax.rem(i, 3)
        pltpu.make_async_copy(xw_hbm.at[cols_v.at[i]], gath_v.at[b], sem).wait()

        @pl.when(i + 2 < SCN)
        def _():
          @pl.when(i > 0)
          def _():
            pltpu.make_async_copy(gath_v.at[b], acc.at[rows_v.at[i - 1]],
                                  ssem).wait()
          pltpu.async_copy(xw_hbm.at[cols_v.at[i + 2]],
                           gath_v.at[lax.rem(i + 2, 3)], sem)

        base = i * C
        for jv in range(C // 16):
          vv = vals_v[pl.ds(base + jv * 16, 16)]
          for l in range(16):
            j = jv * 16 + l
            v = vv[l]
            for f in range(D // 16):
              sl = pl.ds(f * 16, 16)
              gath_v[b, j, sl] = gath_v[b, j, sl] * v
        pltpu.async_copy(gath_v.at[b], acc.at[rows_v.at[i]], ssem, add=True)
        return carry2

      lax.fori_loop(0, SCN, chunk, 0)
      pltpu.make_async_copy(gath_v.at[0], acc.at[rows_v.at[0]], ssem).wait()
      pltpu.make_async_copy(gath_v.at[1], acc.at[rows_v.at[0]], ssem).wait()
      pltpu.make_async_copy(gath_v.at[2], acc.at[rows_v.at[0]], ssem).wait()
      return carry

    lax.fori_loop(0, nsc, super_chunk, 0)

  run(xw_ui, ui_cols3, ui_rows3, ui_vals, acc, UI_PW, UI_SC)
  plsc.subcore_barrier()
  pltpu.sync_copy(acc.at[pl.ds(s * ROWS_UI, ROWS_UI)],
                  out_ui.at[c].at[pl.ds(s * ROWS_UI, ROWS_UI)])
  plsc.subcore_barrier()

  pltpu.sync_copy(zeros_hbm.at[pl.ds(0, ROWS_UU)],
                  acc.at[pl.ds(s * ROWS_UU, ROWS_UU)])
  plsc.subcore_barrier()
  run(xw_uu, uu_cols3, uu_rows3, uu_vals, acc, UU_PW, UU_SC)
  plsc.subcore_barrier()
  pltpu.sync_copy(acc.at[pl.ds(s * ROWS_UU, ROWS_UU)],
                  out_uu.at[c].at[pl.ds(s * ROWS_UU, ROWS_UU)])


_spmm = pl.kernel(
    _spmm_body,
    out_type=[jax.ShapeDtypeStruct((NC, NP, D), jnp.float32),
              jax.ShapeDtypeStruct((NC, NUP, D), jnp.float32)],
    mesh=plsc.VectorSubcoreMesh(core_axis_name="c", subcore_axis_name="s"),
    scratch_types=[
        pltpu.VMEM((SCN, C), jnp.int32),
        pltpu.VMEM((SCN, C), jnp.int32),
        pltpu.VMEM((SCE,), jnp.float32),
        pltpu.VMEM((3, C, D), jnp.float32),
        pltpu.VMEM_SHARED((NP, D), jnp.float32),
        pltpu.SemaphoreType.DMA,
        pltpu.SemaphoreType.DMA,
    ],
)


def _mm_body(a_ref, w_ref, o_ref):
  o_ref[...] = jnp.dot(a_ref[...], w_ref[...], preferred_element_type=jnp.float32)


def _mm(a, w, bm):
  m, k = a.shape
  n = w.shape[1]
  return pl.pallas_call(
      _mm_body,
      grid=(m // bm,),
      in_specs=[pl.BlockSpec((bm, k), lambda i: (i, 0)),
                pl.BlockSpec((k, n), lambda i: (0, 0))],
      out_specs=pl.BlockSpec((bm, n), lambda i: (i, 0)),
      out_shape=jax.ShapeDtypeStruct((m, n), jnp.float32),
  )(a, w)


def _relu_mm_body(p_ref, w_ref, o_ref):
  x = jnp.maximum(p_ref[0] + p_ref[1], 0.0)
  o_ref[...] = jnp.dot(x, w_ref[...], preferred_element_type=jnp.float32)


def _relu_mm(p, w, bm):
  _, m, k = p.shape
  n = w.shape[1]
  return pl.pallas_call(
      _relu_mm_body,
      grid=(m // bm,),
      in_specs=[pl.BlockSpec((NC, bm, k), lambda i: (0, i, 0)),
                pl.BlockSpec((k, n), lambda i: (0, 0))],
      out_specs=pl.BlockSpec((bm, n), lambda i: (i, 0)),
      out_shape=jax.ShapeDtypeStruct((m, n), jnp.float32),
  )(p, w)


def _combine_body(p_ref, o_ref):
  o_ref[...] = p_ref[0] + p_ref[1]


def _combine(p, bm):
  _, m, k = p.shape
  return pl.pallas_call(
      _combine_body,
      grid=(m // bm,),
      in_specs=[pl.BlockSpec((NC, bm, k), lambda i: (0, i, 0))],
      out_specs=pl.BlockSpec((bm, k), lambda i: (i, 0)),
      out_shape=jax.ShapeDtypeStruct((m, k), jnp.float32),
  )(p)


def _dec_body(a_ref, b_ref, o_ref):
  o_ref[...] = lax.dot_general(a_ref[...], b_ref[...],
                               (((1,), (1,)), ((), ())),
                               preferred_element_type=jnp.float32)


def _dec(a, b, bm):
  m = a.shape[0]
  n = b.shape[0]
  k = a.shape[1]
  return pl.pallas_call(
      _dec_body,
      grid=(m // bm,),
      in_specs=[pl.BlockSpec((bm, k), lambda i: (i, 0)),
                pl.BlockSpec((n, k), lambda i: (0, 0))],
      out_specs=pl.BlockSpec((bm, n), lambda i: (i, 0)),
      out_shape=jax.ShapeDtypeStruct((m, n), jnp.float32),
  )(a, b)


def kernel(adj_ui_index, adj_ui_values, adj_uu_index, adj_uu_values,
           emb, W1_ui, W2_ui, W1_uu, W2_uu):
  ui_rows3 = adj_ui_index[0].reshape(NW, UI_SC, SCN, C)
  ui_cols3 = adj_ui_index[1].reshape(NW, UI_SC, SCN, C)
  uu_rows3 = adj_uu_index[0].reshape(NW, UU_SC, SCN, C)
  uu_cols3 = adj_uu_index[1].reshape(NW, UU_SC, SCN, C)
  zeros = jnp.zeros((ROWS_UI, D), jnp.float32)

  xw_ui = _mm(emb, W1_ui, 1000)
  xw_uu = _mm(emb[:NUP], W1_uu, 1024)
  p_ui, p_uu = _spmm(xw_ui, xw_uu, zeros,
                     ui_cols3, ui_rows3, adj_ui_values,
                     uu_cols3, uu_rows3, adj_uu_values)

  y_ui = _relu_mm(p_ui, W2_ui, 1264)
  y_uu = _relu_mm(p_uu, W2_uu, 1024)
  q_ui, q_uu = _spmm(y_ui, y_uu, zeros,
                     ui_cols3, ui_rows3, adj_ui_values,
                     uu_cols3, uu_rows3, adj_uu_values)

  h2_ui = _combine(q_ui, 1264)[:N]
  h2_uu = _combine(q_uu, 1024)[:NU]

  mu_ui = h2_ui[:, :DH2]
  logvar_ui = h2_ui[:, DH2:]
  mu_uu = h2_uu[:, :DH2]
  logvar_uu = h2_uu[:, DH2:]

  dec_ui = _dec(mu_ui[:NU], mu_ui[NU:], 400)
  dec_uu = _dec(mu_uu, mu_uu, 400)
  return (dec_ui, mu_ui, logvar_ui, dec_uu, mu_uu, logvar_uu)
